# baseline (device time: 8868 ns/iter reference)
import jax
import jax.numpy as jnp
from jax import lax
from jax.experimental import pallas as pl
from jax.experimental.pallas import tpu as pltpu

N_DEV = 4


def kernel(x):
    m, n = x.shape

    def body(x_ref, out_ref, comm_ref, send_ref, send_sems, recv_sems):
        my = lax.axis_index("i")


        logs = jnp.log(x_ref[...])

        tot = logs
        half = m
        while half > 1:
            half //= 2
            tot = tot[:half, :] + tot[half:, :]
        send_ref[...] = tot

        def mk(s, r):
            return pltpu.make_async_remote_copy(
                src_ref=send_ref,
                dst_ref=comm_ref.at[s],
                send_sem=send_sems.at[r],
                recv_sem=recv_sems.at[s],
                device_id=(r,),
                device_id_type=pl.DeviceIdType.MESH,
            )

        for s in range(N_DEV - 1):
            @pl.when(my == s)
            def _(s=s):
                for r in range(s + 1, N_DEV):
                    mk(s, r).start()

        row = lax.broadcasted_iota(jnp.int32, (m, m), 0)
        col = lax.broadcasted_iota(jnp.int32, (m, m), 1)
        tri = (row >= col).astype(jnp.bfloat16)
        cums = lax.dot_general(
            tri,
            logs.astype(jnp.bfloat16),
            (((1,), (0,)), ((), ())),
            preferred_element_type=jnp.float32,
        )
        local_incl = jnp.exp(cums)

        for s in range(N_DEV - 1):
            @pl.when(my > s)
            def _(s=s):
                mk(s, (s + 1) % N_DEV).wait_recv()

        zero = jnp.zeros((1, n), jnp.float32)
        log_prefix = zero
        for s in range(N_DEV - 1):
            log_prefix = log_prefix + jnp.where(my > s, comm_ref[s], zero)
        out_ref[...] = local_incl * jnp.exp(log_prefix)

        for s in range(N_DEV - 1):
            @pl.when(my == s)
            def _(s=s):
                for r in range(s + 1, N_DEV):
                    mk(s, r).wait_send()

    return pl.pallas_call(
        body,
        out_shape=jax.ShapeDtypeStruct((m, n), jnp.float32),
        in_specs=[pl.BlockSpec(memory_space=pltpu.VMEM)],
        out_specs=pl.BlockSpec(memory_space=pltpu.VMEM),
        scratch_shapes=[
            pltpu.VMEM((N_DEV - 1, 1, n), jnp.float32),
            pltpu.VMEM((1, n), jnp.float32),
            pltpu.SemaphoreType.DMA((N_DEV,)),
            pltpu.SemaphoreType.DMA((N_DEV,)),
        ],
    )(x)


# device time: 4741 ns/iter; 1.8705x vs baseline; 1.8705x over previous
import jax
import jax.numpy as jnp
from jax import lax
from jax.experimental import pallas as pl
from jax.experimental.pallas import tpu as pltpu

N_DEV = 4


def kernel(x):
    m, n = x.shape

    def body(x_ref, out_ref, comm_ref, send_ref, send_sems, recv_sems):
        my = lax.axis_index("i")

        barrier_sem = pltpu.get_barrier_semaphore()
        for s in range(N_DEV - 1):
            for r in range(s + 1, N_DEV):
                @pl.when(my == r)
                def _(s=s):
                    pl.semaphore_signal(
                        barrier_sem, inc=2,
                        device_id=(s,), device_id_type=pl.DeviceIdType.MESH,
                    )

        xv = x_ref[...]
        tot = xv
        half = m
        while half > 1:
            half //= 2
            tot = tot[:half, :] * tot[half:, :]
        send_ref[...] = jnp.log(tot)

        def mk(s, r):
            return pltpu.make_async_remote_copy(
                src_ref=send_ref,
                dst_ref=comm_ref.at[s],
                send_sem=send_sems.at[r],
                recv_sem=recv_sems.at[s],
                device_id=(r,),
                device_id_type=pl.DeviceIdType.MESH,
            )

        for s in range(N_DEV - 1):
            @pl.when(my == s)
            def _(s=s):
                pl.semaphore_wait(barrier_sem, N_DEV - 1 - s)
                for r in range(s + 1, N_DEV):
                    mk(s, r).start()

        logs = jnp.log(xv)
        row = lax.broadcasted_iota(jnp.int32, (m, m), 0)
        col = lax.broadcasted_iota(jnp.int32, (m, m), 1)
        tri = (row >= col).astype(jnp.bfloat16)
        cums = lax.dot_general(
            tri,
            logs.astype(jnp.bfloat16),
            (((1,), (0,)), ((), ())),
            preferred_element_type=jnp.float32,
        )
        local_incl = jnp.exp(cums)

        for s in range(N_DEV - 1):
            @pl.when(my > s)
            def _(s=s):
                mk(s, (s + 1) % N_DEV).wait_recv()

        zero = jnp.zeros((1, n), jnp.float32)
        log_prefix = zero
        for s in range(N_DEV - 1):
            log_prefix = log_prefix + jnp.where(my > s, comm_ref[s], zero)
        out_ref[...] = local_incl * jnp.exp(log_prefix)

        for s in range(N_DEV - 1):
            @pl.when(my == s)
            def _(s=s):
                for r in range(s + 1, N_DEV):
                    mk(s, r).wait_send()

    return pl.pallas_call(
        body,
        out_shape=jax.ShapeDtypeStruct((m, n), jnp.float32),
        in_specs=[pl.BlockSpec(memory_space=pltpu.VMEM)],
        out_specs=pl.BlockSpec(memory_space=pltpu.VMEM),
        scratch_shapes=[
            pltpu.VMEM((N_DEV - 1, 1, n), jnp.float32),
            pltpu.VMEM((1, n), jnp.float32),
            pltpu.SemaphoreType.DMA((N_DEV,)),
            pltpu.SemaphoreType.DMA((N_DEV,)),
        ],
        compiler_params=pltpu.CompilerParams(collective_id=0),
    )(x)
